# Initial kernel scaffold; baseline (speedup 1.0000x reference)
import jax
import jax.numpy as jnp
from jax.experimental import pallas as pl


def kernel(attn_weights):
    raise NotImplementedError("write your pallas kernel here")


# speedup vs baseline: 8.8175x; 8.8175x over previous
import functools

import jax
import jax.numpy as jnp
from jax.experimental import pallas as pl
from jax.experimental.pallas import tpu as pltpu

_SINK = 4
_RECENT = 64
_KEEP_RATIO = 0.5
_ALPHA = 0.2


def _floor_div(a, b):
    q = jax.lax.div(a, jnp.int32(b))
    r = a - q * b
    return q - jnp.where(r < 0, jnp.int32(1), jnp.int32(0))


def _body(w_ref, mask_ref, *, sink, recent, total_mid_budget, min_budget):
    w = w_ref[0]
    H, L_q, L_kv = w.shape
    middle_len = L_kv - sink - recent
    eps = jnp.float32(1e-8)

    scores = jnp.sum(w, axis=1)
    ent_elem = w * jnp.log(w + eps)
    ent = -jnp.sum(ent_elem, axis=(1, 2), keepdims=False)
    head_entropy = (ent / jnp.float32(L_q)).reshape(H, 1)

    denom = jnp.sum(head_entropy) + eps
    alloc = head_entropy / denom
    budgets = jnp.round(alloc * jnp.float32(total_mid_budget)).astype(jnp.int32)
    budgets = jnp.maximum(budgets, jnp.int32(min_budget))
    current_total = jnp.sum(budgets)
    diff = jnp.int32(total_mid_budget) - current_total
    per_head_adj = _floor_div(diff, H)
    budgets = budgets + per_head_adj
    r = diff - per_head_adj * H
    idx_h = jax.lax.broadcasted_iota(jnp.int32, (H, 1), 0)
    budgets = budgets + jnp.where(idx_h < r, jnp.int32(1), jnp.int32(0))
    budgets = jnp.clip(budgets, jnp.int32(1), jnp.int32(middle_len))

    bits = jax.lax.bitcast_convert_type(scores, jnp.int32)
    col = jax.lax.broadcasted_iota(jnp.int32, (H, L_kv), 1)
    mid_mask = (col >= sink) & (col < L_kv - recent)
    bits_m = jnp.where(mid_mask, bits, jnp.int32(-1))

    def count_ge(t):
        return jnp.sum((bits_m >= t).astype(jnp.int32), axis=1, keepdims=True)

    def val_step(_, carry):
        lo, hi = carry
        mid = lo + jax.lax.div(hi - lo, jnp.int32(2))
        ge = count_ge(mid) >= budgets
        return jnp.where(ge, mid, lo), jnp.where(ge, hi, mid)

    lo0 = jnp.zeros((H, 1), jnp.int32)
    hi0 = jnp.full((H, 1), jnp.int32(0x41000001))
    lo, _ = jax.lax.fori_loop(0, 31, val_step, (lo0, hi0))
    thresh = lo

    cnt_gt = count_ge(thresh + 1)
    m = budgets - cnt_gt
    tie = mid_mask & (bits == thresh)

    def tie_count_lt(c):
        return jnp.sum((tie & (col < c)).astype(jnp.int32), axis=1, keepdims=True)

    def idx_step(_, carry):
        lo2, hi2 = carry
        mid2 = jax.lax.div(lo2 + hi2, jnp.int32(2))
        ge = tie_count_lt(mid2) >= m
        return jnp.where(ge, lo2, mid2), jnp.where(ge, mid2, hi2)

    lo2_0 = jnp.zeros((H, 1), jnp.int32)
    hi2_0 = jnp.full((H, 1), jnp.int32(L_kv))
    _, cstar = jax.lax.fori_loop(0, 12, idx_step, (lo2_0, hi2_0))

    keep = (bits_m > thresh) | (tie & (col < cstar))
    mask = keep | (col < sink) | (col >= L_kv - recent)
    mask_ref[0] = mask.astype(mask_ref.dtype)


def kernel(attn_weights):
    B, H, L_q, L_kv = attn_weights.shape
    sink, recent = _SINK, _RECENT
    n_protected = min(sink + recent, L_kv)
    middle_len = L_kv - n_protected
    if middle_len <= 0:
        return jnp.ones((B, H, L_kv), dtype=bool)
    total_keep = int(L_kv * _KEEP_RATIO)
    middle_budget = max(total_keep - n_protected, 0)
    total_mid_budget = middle_budget * H
    min_budget = max(int(middle_len * _KEEP_RATIO * _ALPHA), 1)

    body = functools.partial(
        _body,
        sink=sink,
        recent=recent,
        total_mid_budget=total_mid_budget,
        min_budget=min_budget,
    )
    out = pl.pallas_call(
        body,
        grid=(B,),
        in_specs=[pl.BlockSpec((1, H, L_q, L_kv), lambda b: (b, 0, 0, 0))],
        out_specs=pl.BlockSpec((1, H, L_kv), lambda b: (b, 0, 0)),
        out_shape=jax.ShapeDtypeStruct((B, H, L_kv), jnp.bool_),
    )(attn_weights)
    return out
